# Initial kernel scaffold; baseline (speedup 1.0000x reference)
import jax
import jax.numpy as jnp
from jax.experimental import pallas as pl


def kernel(nodes, indices):
    raise NotImplementedError("write your pallas kernel here")


# speedup vs baseline: 1.0818x; 1.0818x over previous
import functools

import jax
import jax.numpy as jnp
from jax import lax
from jax.experimental import pallas as pl
from jax.experimental.pallas import tpu as pltpu
from jax.experimental.pallas import tpu_sc as plsc

N = 100000
D = 128
SEG = 1024
CHUNK = 128
NW = 32
NCHUNKS = (N + CHUNK - 1) // CHUNK
TAIL = N - (NCHUNKS - 1) * CHUNK
KMAX = (NCHUNKS + NW - 1) // NW
GROUPS = D // 16
ROWS_PER_SUB = SEG // 16


def _zero_rows(buf, nrows):
    z = jnp.zeros((16,), jnp.float32)

    def body(r, _):
        for g in range(GROUPS):
            buf[r, pl.ds(g * 16, 16)] = z
        return 0

    lax.fori_loop(0, nrows, body, 0)


def _make_pass(first: bool):
    mesh = plsc.VectorSubcoreMesh(core_axis_name="c", subcore_axis_name="s")

    out_type = [jax.ShapeDtypeStruct((2, SEG, D), jnp.float32)]
    if first:
        out_type.append(jax.ShapeDtypeStruct((NW, D), jnp.float32))
    else:
        out_type.append(jax.ShapeDtypeStruct((NW, D), jnp.float32))

    scratch = [
        pltpu.VMEM((CHUNK, D), jnp.float32),
        pltpu.VMEM((CHUNK,), jnp.int32),
        pltpu.VMEM((TAIL, D), jnp.float32),
        pltpu.VMEM((TAIL,), jnp.int32),
        pltpu.VMEM((D,), jnp.float32),
        pltpu.VMEM((ROWS_PER_SUB, D), jnp.float32),
        pltpu.VMEM_SHARED((SEG, D), jnp.float32),
        pltpu.SemaphoreType.DMA,
    ]
    if not first:
        scratch = [
            pltpu.VMEM((CHUNK, D), jnp.float32),
            pltpu.VMEM((CHUNK, D), jnp.float32),
            pltpu.VMEM((TAIL, D), jnp.float32),
            pltpu.VMEM((TAIL, D), jnp.float32),
            pltpu.VMEM((D,), jnp.float32),
        ] + scratch

    def body(*refs):
        if first:
            (nodes_hbm, idx_hbm, a_out, aux_out,
             nbuf, ibuf, nbuf_t, ibuf_t, acc, stage, shared, sem) = refs
            gbuf = ybuf = gbuf_t = ybuf_t = mbuf = s_hbm = None
        else:
            (nodes_hbm, idx_hbm, s_hbm, m_hbm, a_out, aux_out,
             gbuf, ybuf, gbuf_t, ybuf_t, mbuf,
             nbuf, ibuf, nbuf_t, ibuf_t, acc, stage, shared, sem) = refs

        cid = lax.axis_index("c")
        sid = lax.axis_index("s")
        wid = sid * 2 + cid

        _zero_rows(stage, ROWS_PER_SUB)
        pltpu.sync_copy(stage, shared.at[pl.ds(sid * ROWS_PER_SUB, ROWS_PER_SUB)])
        z = jnp.zeros((16,), jnp.float32)
        for g in range(GROUPS):
            acc[pl.ds(g * 16, 16)] = z
        if not first:
            pltpu.sync_copy(m_hbm, mbuf)
        plsc.subcore_barrier()

        if not first:
            mv = [mbuf[pl.ds(g * 16, 16)] for g in range(GROUPS)]

        def chunk_body(nb, ib, gb, yb, nrows, row0):
            pltpu.sync_copy(nodes_hbm.at[pl.ds(row0, nrows)], nb)
            pltpu.sync_copy(idx_hbm.at[pl.ds(row0, nrows)], ib)
            if first:
                def rbody(r, _):
                    for g in range(GROUPS):
                        sl = pl.ds(g * 16, 16)
                        n = nb[r, sl]
                        acc[sl] = jnp.maximum(acc[sl], jnp.abs(n))
                    return 0

                lax.fori_loop(0, nrows, rbody, 0)
                pltpu.sync_copy(nb, shared.at[ib], add=True)
            else:
                pltpu.async_copy(s_hbm.at[ib], gb, sem).wait()

                def rbody(r, _):
                    for g in range(GROUPS):
                        sl = pl.ds(g * 16, 16)
                        n = nb[r, sl]
                        e = jnp.exp(n * gb[r, sl] - mv[g])
                        plsc.addupdate(acc.at[sl], e)
                        yb[r, sl] = e * n
                    return 0

                lax.fori_loop(0, nrows, rbody, 0)
                pltpu.sync_copy(yb, shared.at[ib], add=True)

        def outer(kk, _):
            j = kk * NW + wid

            @pl.when(j < NCHUNKS - 1)
            def _():
                chunk_body(nbuf, ibuf, gbuf, ybuf, CHUNK, j * CHUNK)

            @pl.when(j == NCHUNKS - 1)
            def _():
                chunk_body(nbuf_t, ibuf_t, gbuf_t, ybuf_t, TAIL, j * CHUNK)

            return 0

        lax.fori_loop(0, KMAX, outer, 0)

        pltpu.sync_copy(acc, aux_out.at[wid])
        plsc.subcore_barrier()
        sl = pl.ds(sid * ROWS_PER_SUB, ROWS_PER_SUB)
        pltpu.sync_copy(shared.at[sl], stage)
        pltpu.sync_copy(stage, a_out.at[cid, sl])

    out_type = tuple(out_type)
    kern = functools.partial(
        pl.kernel, mesh=mesh, out_type=out_type, scratch_types=tuple(scratch)
    )(body)
    return kern


_pass1 = _make_pass(first=True)
_passk = _make_pass(first=False)


def kernel(nodes, indices):
    idx = indices.astype(jnp.int32)
    a1, absp = _pass1(nodes, idx)
    absn = jnp.max(absp, axis=0)
    sigma = (a1[0] + a1[1]) * jnp.float32(1.0 / N)
    s_tab = sigma
    for _ in range(2):
        m = absn * jnp.max(jnp.abs(s_tab), axis=0)
        ak, den = _passk(nodes, idx, s_tab, m)
        sigma = (ak[0] + ak[1]) / jnp.sum(den, axis=0)
        s_tab = s_tab + sigma
    return sigma


# speedup vs baseline: 4.2885x; 3.9644x over previous
import jax
import jax.numpy as jnp
from jax import lax
from jax.experimental import pallas as pl
from jax.experimental.pallas import tpu as pltpu
from jax.experimental.pallas import tpu_sc as plsc

import functools

N = 100000
D = 128
SEG = 1024
CHUNK = 128
NW = 32
NCHUNKS = (N + CHUNK - 1) // CHUNK
NFULL = NCHUNKS - 1
TAIL = N - NFULL * CHUNK
TAIL_WID = NFULL % NW
KMAX = (NFULL + NW - 1) // NW
GROUPS = D // 16
ROWS_PER_SUB = SEG // 16


def _make_pass(first: bool):
    mesh = plsc.VectorSubcoreMesh(core_axis_name="c", subcore_axis_name="s")

    out_type = (
        jax.ShapeDtypeStruct((2, SEG, D), jnp.float32),
        jax.ShapeDtypeStruct((NW, D), jnp.float32),
    )

    scratch = []
    if not first:
        scratch += [
            pltpu.VMEM((2, CHUNK, D), jnp.float32),
            pltpu.VMEM((2, CHUNK, D), jnp.float32),
            pltpu.VMEM((D,), jnp.float32),
        ]
    scratch += [
        pltpu.VMEM((2, CHUNK, D), jnp.float32),
        pltpu.VMEM((KMAX, CHUNK), jnp.int32),
        pltpu.VMEM((TAIL,), jnp.int32),
        pltpu.VMEM((D,), jnp.float32),
        pltpu.VMEM((ROWS_PER_SUB, D), jnp.float32),
        pltpu.VMEM_SHARED((SEG, D), jnp.float32),
        pltpu.SemaphoreType.DMA,
        pltpu.SemaphoreType.DMA,
        pltpu.SemaphoreType.DMA,
        pltpu.SemaphoreType.DMA,
        pltpu.SemaphoreType.DMA,
        pltpu.SemaphoreType.DMA,
        pltpu.SemaphoreType.DMA,
    ]

    def body(*refs):
        if first:
            (nodes_hbm, idx_hbm, a_out, aux_out,
             nbuf, ibig, ibuf_t, acc, stage, shared,
             sem_i, sem_n0, sem_n1, sem_g0, sem_g1, sem_y0, sem_y1) = refs
            gbuf = ybuf = mbuf = s_hbm = None
        else:
            (nodes_hbm, idx_hbm, s_hbm, m_hbm, a_out, aux_out,
             gbuf, ybuf, mbuf,
             nbuf, ibig, ibuf_t, acc, stage, shared,
             sem_i, sem_n0, sem_n1, sem_g0, sem_g1, sem_y0, sem_y1) = refs
        sem_n = (sem_n0, sem_n1)
        sem_g = (sem_g0, sem_g1)
        sem_y = (sem_y0, sem_y1)

        cid = lax.axis_index("c")
        sid = lax.axis_index("s")
        wid = sid * 2 + cid
        nfull_w = (NFULL - wid + NW - 1) // NW

        z = jnp.zeros((16,), jnp.float32)

        def zrow(r, _):
            for g in range(GROUPS):
                stage[r, pl.ds(g * 16, 16)] = z
            return 0

        lax.fori_loop(0, ROWS_PER_SUB, zrow, 0)
        pltpu.sync_copy(stage, shared.at[pl.ds(sid * ROWS_PER_SUB, ROWS_PER_SUB)])
        for g in range(GROUPS):
            acc[pl.ds(g * 16, 16)] = z
        if not first:
            pltpu.sync_copy(m_hbm, mbuf)

        def iload(kk, _):
            @pl.when(kk < nfull_w)
            def _():
                j = kk * NW + wid
                pltpu.async_copy(idx_hbm.at[pl.ds(j * CHUNK, CHUNK)],
                                 ibig.at[kk], sem_i)
            return 0

        lax.fori_loop(0, KMAX, iload, 0)

        def idrain(kk, _):
            @pl.when(kk < nfull_w)
            def _():
                pltpu.make_async_copy(idx_hbm.at[pl.ds(0, CHUNK)],
                                      ibig.at[kk], sem_i).wait()
            return 0

        lax.fori_loop(0, KMAX, idrain, 0)

        plsc.subcore_barrier()

        if not first:
            mv = [mbuf[pl.ds(g * 16, 16)] for g in range(GROUPS)]

        def issue(kx, px):
            j = kx * NW + wid
            if first:
                @pl.when(kx >= 2)
                def _():
                    pltpu.make_async_copy(nodes_hbm.at[pl.ds(0, CHUNK)],
                                          nbuf.at[px], sem_y[px]).wait()
            pltpu.async_copy(nodes_hbm.at[pl.ds(j * CHUNK, CHUNK)],
                             nbuf.at[px], sem_n[px])
            if not first:
                pltpu.async_copy(s_hbm.at[ibig.at[kx]], gbuf.at[px], sem_g[px])

        sls = [pl.ds(g * 16, 16) for g in range(GROUPS)]
        zs = tuple(z for _ in range(GROUPS))

        def compute(p):
            if first:

                @plsc.parallel_loop(0, CHUNK, 1, carry=zs)
                def am(r, c):
                    return tuple(
                        jnp.maximum(c[g], jnp.abs(nbuf[p, r, sls[g]]))
                        for g in range(GROUPS))

                for g in range(GROUPS):
                    acc[sls[g]] = jnp.maximum(acc[sls[g]], am[g])
            else:

                @plsc.parallel_loop(0, CHUNK, 1, carry=zs)
                def dn(r, c):
                    outs = []
                    for g in range(GROUPS):
                        n = nbuf[p, r, sls[g]]
                        e = jnp.exp(n * gbuf[p, r, sls[g]] - mv[g])
                        ybuf[p, r, sls[g]] = e * n
                        outs.append(c[g] + e)
                    return tuple(outs)

                for g in range(GROUPS):
                    plsc.addupdate(acc.at[sls[g]], dn[g])

        def step(kk, p):
            @pl.when(kk < nfull_w)
            def _():
                @pl.when(kk + 1 < nfull_w)
                def _():
                    issue(kk + 1, 1 - p)

                pltpu.make_async_copy(nodes_hbm.at[pl.ds(0, CHUNK)],
                                      nbuf.at[p], sem_n[p]).wait()
                if not first:
                    pltpu.make_async_copy(nodes_hbm.at[pl.ds(0, CHUNK)],
                                          gbuf.at[p], sem_g[p]).wait()

                    @pl.when(kk >= 2)
                    def _():
                        pltpu.make_async_copy(nodes_hbm.at[pl.ds(0, CHUNK)],
                                              ybuf.at[p], sem_y[p]).wait()

                compute(p)
                src = nbuf.at[p] if first else ybuf.at[p]
                pltpu.async_copy(src, shared.at[ibig.at[kk]], sem_y[p], add=True)

        @pl.when(0 < nfull_w)
        def _():
            issue(0, 0)

        def outer(kk2, _):
            step(kk2 * 2, 0)
            step(kk2 * 2 + 1, 1)
            return 0

        lax.fori_loop(0, (KMAX + 1) // 2, outer, 0)

        dr = (nbuf, nbuf) if first else (ybuf, ybuf)

        @pl.when(nfull_w >= 1)
        def _():
            pltpu.make_async_copy(nodes_hbm.at[pl.ds(0, CHUNK)], dr[0].at[0],
                                  sem_y[0]).wait()

        @pl.when(nfull_w >= 2)
        def _():
            pltpu.make_async_copy(nodes_hbm.at[pl.ds(0, CHUNK)], dr[1].at[1],
                                  sem_y[1]).wait()

        @pl.when(wid == TAIL_WID)
        def _():
            row0 = NFULL * CHUNK
            tsl = pl.ds(0, TAIL)
            pltpu.sync_copy(nodes_hbm.at[pl.ds(row0, TAIL)], nbuf.at[0, tsl])
            pltpu.sync_copy(idx_hbm.at[pl.ds(row0, TAIL)], ibuf_t)
            if first:

                @plsc.parallel_loop(0, TAIL, 1, carry=zs)
                def am(r, c):
                    return tuple(
                        jnp.maximum(c[g], jnp.abs(nbuf[0, r, sls[g]]))
                        for g in range(GROUPS))

                for g in range(GROUPS):
                    acc[sls[g]] = jnp.maximum(acc[sls[g]], am[g])
                pltpu.sync_copy(nbuf.at[0, tsl], shared.at[ibuf_t], add=True)
            else:
                pltpu.async_copy(s_hbm.at[ibuf_t], gbuf.at[0, tsl], sem_g0).wait()

                @plsc.parallel_loop(0, TAIL, 1, carry=zs)
                def dn(r, c):
                    outs = []
                    for g in range(GROUPS):
                        n = nbuf[0, r, sls[g]]
                        e = jnp.exp(n * gbuf[0, r, sls[g]] - mv[g])
                        ybuf[0, r, sls[g]] = e * n
                        outs.append(c[g] + e)
                    return tuple(outs)

                for g in range(GROUPS):
                    plsc.addupdate(acc.at[sls[g]], dn[g])
                pltpu.sync_copy(ybuf.at[0, tsl], shared.at[ibuf_t], add=True)

        pltpu.sync_copy(acc, aux_out.at[wid])
        plsc.subcore_barrier()
        osl = pl.ds(sid * ROWS_PER_SUB, ROWS_PER_SUB)
        pltpu.sync_copy(shared.at[osl], stage)
        pltpu.sync_copy(stage, a_out.at[cid, osl])

    return functools.partial(
        pl.kernel, mesh=mesh, out_type=out_type, scratch_types=tuple(scratch)
    )(body)


_pass1 = _make_pass(first=True)
_passk = _make_pass(first=False)


def kernel(nodes, indices):
    idx = indices.astype(jnp.int32)
    a1, absp = _pass1(nodes, idx)
    absn = jnp.max(absp, axis=0)
    sigma = (a1[0] + a1[1]) * jnp.float32(1.0 / N)
    s_tab = sigma
    for _ in range(2):
        m = absn * jnp.max(jnp.abs(s_tab), axis=0)
        ak, den = _passk(nodes, idx, s_tab, m)
        sigma = (ak[0] + ak[1]) / jnp.sum(den, axis=0)
        s_tab = s_tab + sigma
    return sigma
